# baseline (device time: 88327 ns/iter reference)
import jax
import jax.numpy as jnp
from jax import lax
from jax.experimental import pallas as pl
from jax.experimental.pallas import tpu as pltpu

N_DEV = 8
T_A = 16
NB_B = 16
TOT = T_A + NB_B
BN_A = 512
BN_B = 512
NBUFB = 4
NREM = N_DEV - 1


def kernel(x, w_mat, scale_x, scale_w):
    m_global, k_shard = x.shape
    k_global, n = w_mat.shape
    m_per = m_global // N_DEV
    bks = k_global // N_DEV

    def body(x_ref, w_ref, sx_ref, sw_ref, out_ref,
             xs_ref, comm_ref, a_ref, wbufa_ref, wbufb_ref, xstg_ref,
             send_sems, recv_sems, wasems, wbsems, xsems):
        t = pl.program_id(0)
        my = lax.axis_index("i")

        def peer_rdma(d, slot_dst, slot_sem):
            return pltpu.make_async_remote_copy(
                src_ref=xs_ref.at[pl.ds(d * m_per, m_per), :],
                dst_ref=comm_ref.at[slot_dst],
                send_sem=send_sems.at[slot_sem],
                recv_sem=recv_sems.at[slot_dst],
                device_id=(d,),
                device_id_type=pl.DeviceIdType.MESH,
            )

        def wcopy_a(b):
            return pltpu.make_async_copy(
                w_ref.at[pl.ds(my * bks, bks), pl.ds(b * BN_A, BN_A)],
                wbufa_ref.at[lax.rem(b, 2)],
                wasems.at[lax.rem(b, 2)],
            )

        def wsub_b(b, jj):
            sr = lax.rem(my + 1 + jj, N_DEV)
            slot = lax.rem(b, NBUFB)
            return pltpu.make_async_copy(
                w_ref.at[pl.ds(sr * bks, bks), pl.ds(b * BN_B, BN_B)],
                wbufb_ref.at[slot, pl.ds(jj * bks, bks), :],
                wbsems.at[slot, jj],
            )

        def wstart_b(b):
            for jj in range(NREM):
                wsub_b(b, jj).start()

        def xcopy(j):
            return pltpu.make_async_copy(
                x_ref.at[pl.ds(j * m_per, m_per), :],
                xstg_ref.at[j % 2],
                xsems.at[j % 2],
            )

        @pl.when(t == 0)
        def _():
            xcopy(0).start()
            xcopy(1).start()
            for j in range(N_DEV):
                xcopy(j).wait()
                xs_ref[pl.ds(j * m_per, m_per), :] = (
                    xstg_ref[j % 2].astype(jnp.float8_e4m3fn))
                if j + 2 < N_DEV:
                    xcopy(j + 2).start()

                @pl.when(j == my)
                def _():
                    comm_ref[my] = xs_ref[pl.ds(j * m_per, m_per), :]
                    a_ref[:, pl.ds(0, bks)] = (
                        comm_ref[my].astype(jnp.bfloat16))

                @pl.when(j != my)
                def _():
                    peer_rdma(j, my, j).start()

            wcopy_a(0).start()
            wcopy_a(1).start()
            for b in range(NBUFB - 1):
                wstart_b(b)

        @pl.when(t < T_A)
        def _():
            wcopy_a(t).wait()
            wba = wbufa_ref[lax.rem(t, 2)].astype(jnp.bfloat16)
            partial = jnp.dot(a_ref[:, pl.ds(0, bks)], wba,
                              preferred_element_type=jnp.float32)
            out_ref[:, pl.ds(t * BN_A, BN_A)] = partial

            @pl.when(t + 2 < T_A)
            def _():
                wcopy_a(t + 2).start()

        @pl.when(t == T_A)
        def _():
            for jj in range(NREM):
                s = lax.rem(my + 1 + jj, N_DEV)
                peer_rdma(my, s, s).wait_recv()
                a_ref[:, pl.ds((1 + jj) * bks, bks)] = (
                    comm_ref[s].astype(jnp.bfloat16))

        @pl.when(t >= T_A)
        def _():
            i = t - T_A
            for jj in range(NREM):
                wsub_b(i, jj).wait()
            wbb = wbufb_ref[lax.rem(i, NBUFB)].astype(jnp.bfloat16)
            partial = jnp.dot(a_ref[:, pl.ds(bks, NREM * bks)], wbb,
                              preferred_element_type=jnp.float32)
            s = sx_ref[0] * sw_ref[0]
            prev = out_ref[:, pl.ds(i * BN_B, BN_B)]
            out_ref[:, pl.ds(i * BN_B, BN_B)] = jnp.maximum(
                (prev + partial) * s, 0.0)

            @pl.when(i + NBUFB - 1 < NB_B)
            def _():
                wstart_b(i + NBUFB - 1)

        @pl.when(t == TOT - 1)
        def _():
            for off in range(1, N_DEV):
                d = lax.rem(my + off, N_DEV)
                peer_rdma(d, my, d).wait_send()

    return pl.pallas_call(
        body,
        grid=(TOT,),
        in_specs=[
            pl.BlockSpec(memory_space=pl.ANY),
            pl.BlockSpec(memory_space=pl.ANY),
            pl.BlockSpec(memory_space=pltpu.SMEM),
            pl.BlockSpec(memory_space=pltpu.SMEM),
        ],
        out_specs=pl.BlockSpec((m_per, n), lambda t: (0, 0),
                               memory_space=pltpu.VMEM),
        out_shape=jax.ShapeDtypeStruct((m_per, n), jnp.float32),
        scratch_shapes=[
            pltpu.VMEM((m_global, k_shard), jnp.float8_e4m3fn),
            pltpu.VMEM((N_DEV, m_per, k_shard), jnp.float8_e4m3fn),
            pltpu.VMEM((m_per, k_global), jnp.bfloat16),
            pltpu.VMEM((2, bks, BN_A), jnp.float32),
            pltpu.VMEM((NBUFB, NREM * bks, BN_B), jnp.float32),
            pltpu.VMEM((2, m_per, k_shard), jnp.float32),
            pltpu.SemaphoreType.DMA((N_DEV,)),
            pltpu.SemaphoreType.DMA((N_DEV,)),
            pltpu.SemaphoreType.DMA((2,)),
            pltpu.SemaphoreType.DMA((NBUFB, NREM)),
            pltpu.SemaphoreType.DMA((2,)),
        ],
        compiler_params=pltpu.CompilerParams(
            dimension_semantics=("arbitrary",),
            vmem_limit_bytes=61 * 1024 * 1024,
        ),
    )(x, w_mat, scale_x, scale_w)


# device time: 82435 ns/iter; 1.0715x vs baseline; 1.0715x over previous
import jax
import jax.numpy as jnp
from jax import lax
from jax.experimental import pallas as pl
from jax.experimental.pallas import tpu as pltpu

N_DEV = 8
T_A = 16
NB_B = 16
TOT = T_A + NB_B
BN_A = 512
BN_B = 512
NBUFB = 4
NREM = N_DEV - 1


def kernel(x, w_mat, scale_x, scale_w):
    m_global, k_shard = x.shape
    k_global, n = w_mat.shape
    m_per = m_global // N_DEV
    bks = k_global // N_DEV

    def body(x_ref, w_ref, sx_ref, sw_ref, out_ref,
             xs_ref, comm_ref, a_ref, wbufa_ref, wbufb_ref, xstg_ref,
             acc_ref, send_sems, recv_sems, wasems, wbsems, xsems, osems):
        t = pl.program_id(0)
        my = lax.axis_index("i")

        def peer_rdma(d, slot_dst, slot_sem):
            return pltpu.make_async_remote_copy(
                src_ref=xs_ref.at[pl.ds(d * m_per, m_per), :],
                dst_ref=comm_ref.at[slot_dst],
                send_sem=send_sems.at[slot_sem],
                recv_sem=recv_sems.at[slot_dst],
                device_id=(d,),
                device_id_type=pl.DeviceIdType.MESH,
            )

        def wcopy_a(b):
            return pltpu.make_async_copy(
                w_ref.at[pl.ds(my * bks, bks), pl.ds(b * BN_A, BN_A)],
                wbufa_ref.at[lax.rem(b, 2)],
                wasems.at[lax.rem(b, 2)],
            )

        def wsub_b(b, jj):
            sr = lax.rem(my + 1 + jj, N_DEV)
            slot = lax.rem(b, NBUFB)
            return pltpu.make_async_copy(
                w_ref.at[pl.ds(sr * bks, bks), pl.ds(b * BN_B, BN_B)],
                wbufb_ref.at[slot, pl.ds(jj * bks, bks), :],
                wbsems.at[slot, jj],
            )

        def wstart_b(b):
            for jj in range(NREM):
                wsub_b(b, jj).start()

        def xcopy(j):
            return pltpu.make_async_copy(
                x_ref.at[pl.ds(j * m_per, m_per), :],
                xstg_ref.at[j % 2],
                xsems.at[j % 2],
            )

        @pl.when(t == 0)
        def _():
            xcopy(0).start()
            xcopy(1).start()
            for j in range(N_DEV):
                xcopy(j).wait()
                xs_ref[pl.ds(j * m_per, m_per), :] = (
                    xstg_ref[j % 2].astype(jnp.float8_e4m3fn))
                if j + 2 < N_DEV:
                    xcopy(j + 2).start()

                @pl.when(j == my)
                def _():
                    comm_ref[my] = xs_ref[pl.ds(j * m_per, m_per), :]
                    a_ref[:, pl.ds(0, bks)] = (
                        comm_ref[my].astype(jnp.bfloat16))

                @pl.when(j != my)
                def _():
                    peer_rdma(j, my, j).start()

            wcopy_a(0).start()
            wcopy_a(1).start()
            for b in range(NBUFB - 1):
                wstart_b(b)

        @pl.when(t < T_A)
        def _():
            wcopy_a(t).wait()
            wba = wbufa_ref[lax.rem(t, 2)].astype(jnp.bfloat16)
            partial = jnp.dot(a_ref[:, pl.ds(0, bks)], wba,
                              preferred_element_type=jnp.float32)
            acc_ref[:, pl.ds(t * BN_A, BN_A)] = partial

            @pl.when(t + 2 < T_A)
            def _():
                wcopy_a(t + 2).start()

        @pl.when(t == T_A)
        def _():
            for jj in range(NREM):
                s = lax.rem(my + 1 + jj, N_DEV)
                peer_rdma(my, s, s).wait_recv()
                a_ref[:, pl.ds((1 + jj) * bks, bks)] = (
                    comm_ref[s].astype(jnp.bfloat16))

        @pl.when(t >= T_A)
        def _():
            i = t - T_A
            for jj in range(NREM):
                wsub_b(i, jj).wait()
            wbb = wbufb_ref[lax.rem(i, NBUFB)].astype(jnp.bfloat16)
            partial = jnp.dot(a_ref[:, pl.ds(bks, NREM * bks)], wbb,
                              preferred_element_type=jnp.float32)
            s = sx_ref[0] * sw_ref[0]
            prev = acc_ref[:, pl.ds(i * BN_B, BN_B)]
            acc_ref[:, pl.ds(i * BN_B, BN_B)] = jnp.maximum(
                (prev + partial) * s, 0.0)
            pltpu.make_async_copy(
                acc_ref.at[:, pl.ds(i * BN_B, BN_B)],
                out_ref.at[:, pl.ds(i * BN_B, BN_B)],
                osems.at[lax.rem(i, NB_B)],
            ).start()

            @pl.when(i + NBUFB - 1 < NB_B)
            def _():
                wstart_b(i + NBUFB - 1)

        @pl.when(t == TOT - 1)
        def _():
            for i2 in range(NB_B):
                pltpu.make_async_copy(
                    acc_ref.at[:, pl.ds(i2 * BN_B, BN_B)],
                    out_ref.at[:, pl.ds(i2 * BN_B, BN_B)],
                    osems.at[i2],
                ).wait()
            for off in range(1, N_DEV):
                d = lax.rem(my + off, N_DEV)
                peer_rdma(d, my, d).wait_send()

    return pl.pallas_call(
        body,
        grid=(TOT,),
        in_specs=[
            pl.BlockSpec(memory_space=pl.ANY),
            pl.BlockSpec(memory_space=pl.ANY),
            pl.BlockSpec(memory_space=pltpu.SMEM),
            pl.BlockSpec(memory_space=pltpu.SMEM),
        ],
        out_specs=pl.BlockSpec(memory_space=pl.ANY),
        out_shape=jax.ShapeDtypeStruct((m_per, n), jnp.float32),
        scratch_shapes=[
            pltpu.VMEM((m_global, k_shard), jnp.float8_e4m3fn),
            pltpu.VMEM((N_DEV, m_per, k_shard), jnp.float8_e4m3fn),
            pltpu.VMEM((m_per, k_global), jnp.bfloat16),
            pltpu.VMEM((2, bks, BN_A), jnp.float32),
            pltpu.VMEM((NBUFB, NREM * bks, BN_B), jnp.float32),
            pltpu.VMEM((2, m_per, k_shard), jnp.float32),
            pltpu.VMEM((m_per, n), jnp.float32),
            pltpu.SemaphoreType.DMA((N_DEV,)),
            pltpu.SemaphoreType.DMA((N_DEV,)),
            pltpu.SemaphoreType.DMA((2,)),
            pltpu.SemaphoreType.DMA((NBUFB, NREM)),
            pltpu.SemaphoreType.DMA((2,)),
            pltpu.SemaphoreType.DMA((NB_B,)),
        ],
        compiler_params=pltpu.CompilerParams(
            dimension_semantics=("arbitrary",),
            vmem_limit_bytes=61 * 1024 * 1024,
        ),
    )(x, w_mat, scale_x, scale_w)
